# baseline (speedup 1.0000x reference)
import functools

import jax
import jax.numpy as jnp
from jax import lax
from jax.experimental import pallas as pl
from jax.experimental.pallas import tpu as pltpu
from jax.experimental.pallas import tpu_sc as plsc

_B, _C, _H, _W, _K = 16, 80, 128, 128, 128
_HW = _H * _W
_ROWS = _B * _C * _H
_GRID = 10
_BLK = _ROWS // _GRID

_LOGIT_CLIP = 9.21024036697585

_NG = _B * _K * 2
_NW = 32
_PER_W = _NG // _NW


_CH = 8
_UNROLL = 128


def _focal_body(x_ref, g_ref, o_ref, acc_ref):
    i = pl.program_id(0)

    @pl.when(i == 0)
    def _():
        acc_ref[0] = 0.0
        acc_ref[1] = 0.0

    def step(j, carry):
        s_acc, n_acc = carry
        x = x_ref[pl.ds(j * _CH, _CH), :]
        g = g_ref[pl.ds(j * _CH, _CH), :]
        xc = jnp.minimum(jnp.maximum(x, -_LOGIT_CLIP), _LOGIT_CLIP)
        t = jnp.exp(-xc)
        u = t + 1.0
        lg = jnp.log(u)
        v = 1.0 / (u * u)
        pos = g == 1.0
        w = 1.0 - g
        w2 = w * w
        m1 = jnp.where(pos, lg * (t * t), (lg + xc) * (w2 * w2))
        s_acc = s_acc + m1 * v
        n_acc = n_acc + jnp.where(pos, 1.0, 0.0)
        return s_acc, n_acc

    z = jnp.zeros((_CH, _W), jnp.float32)
    s_acc, n_acc = lax.fori_loop(0, _BLK // _CH, step, (z, z),
                                 unroll=_UNROLL)
    acc_ref[0] += jnp.sum(s_acc)
    acc_ref[1] += jnp.sum(n_acc)

    @pl.when(i == pl.num_programs(0) - 1)
    def _():
        o_ref[0] = acc_ref[0]
        o_ref[1] = acc_ref[1]


def _focal_sums(hm_pred, hm_gt):
    x = hm_pred.reshape(_ROWS, _W)
    g = hm_gt.reshape(_ROWS, _W)
    return pl.pallas_call(
        _focal_body,
        grid=(_GRID,),
        in_specs=[
            pl.BlockSpec((_BLK, _W), lambda i: (i, 0)),
            pl.BlockSpec((_BLK, _W), lambda i: (i, 0)),
        ],
        out_specs=pl.BlockSpec(memory_space=pltpu.SMEM),
        out_shape=jax.ShapeDtypeStruct((2,), jnp.float32),
        scratch_shapes=[pltpu.SMEM((2,), jnp.float32)],
    )(x, g)


def _sc_gather2(wh_t, rg_t, idx):
    mesh = plsc.VectorSubcoreMesh(core_axis_name="c", subcore_axis_name="s")

    @functools.partial(
        pl.kernel,
        mesh=mesh,
        out_type=(jax.ShapeDtypeStruct((_NG, _W), jnp.float32),
                  jax.ShapeDtypeStruct((_NG, _W), jnp.float32)),
        scratch_types=[
            pltpu.VMEM((_PER_W,), jnp.int32),
            pltpu.VMEM((_PER_W, _W), jnp.float32),
            pltpu.VMEM((_PER_W, _W), jnp.float32),
            pltpu.SemaphoreType.DMA,
            pltpu.SemaphoreType.DMA,
        ],
    )
    def k(wh_hbm, rg_hbm, idx_hbm, ow_hbm, or_hbm,
          idx_v, rw_v, rr_v, sem1, sem2):
        wid = lax.axis_index("s") * 2 + lax.axis_index("c")
        base = wid * _PER_W
        pltpu.sync_copy(idx_hbm.at[pl.ds(base, _PER_W)], idx_v)
        c1 = pltpu.async_copy(wh_hbm.at[idx_v], rw_v, sem1)
        c2 = pltpu.async_copy(rg_hbm.at[idx_v], rr_v, sem2)
        c1.wait()
        pltpu.sync_copy(rw_v, ow_hbm.at[pl.ds(base, _PER_W)])
        c2.wait()
        pltpu.sync_copy(rr_v, or_hbm.at[pl.ds(base, _PER_W)])

    return k(wh_t, rg_t, idx)


def _combine_body(gw_ref, gr_ref, x_ref, tw_ref, tr_ref, m_ref, hm_ref, o_ref):
    onehot = (lax.broadcasted_iota(jnp.int32, (_NG, _W), 1) == x_ref[...]
              ).astype(jnp.float32)
    vw = jnp.sum(gw_ref[...] * onehot, axis=1, keepdims=True)
    vr = jnp.sum(gr_ref[...] * onehot, axis=1, keepdims=True)
    m = m_ref[...]
    contrib = (jnp.abs(vw - tw_ref[...]) * 0.1
               + jnp.abs(vr - tr_ref[...]) * 1.0) * m
    denom = jnp.sum(m) + 1e-4
    hm_loss = hm_ref[0] / jnp.maximum(hm_ref[1], 1.0)
    o_ref[0] = hm_loss + jnp.sum(contrib) / denom


def _combine(gw, gr, xsel, tgt_w, tgt_r, mask2, hm_sums):
    return pl.pallas_call(
        _combine_body,
        in_specs=[
            pl.BlockSpec(memory_space=pltpu.VMEM),
            pl.BlockSpec(memory_space=pltpu.VMEM),
            pl.BlockSpec(memory_space=pltpu.VMEM),
            pl.BlockSpec(memory_space=pltpu.VMEM),
            pl.BlockSpec(memory_space=pltpu.VMEM),
            pl.BlockSpec(memory_space=pltpu.VMEM),
            pl.BlockSpec(memory_space=pltpu.SMEM),
        ],
        out_specs=pl.BlockSpec(memory_space=pltpu.SMEM),
        out_shape=jax.ShapeDtypeStruct((1,), jnp.float32),
    )(gw, gr, xsel, tgt_w, tgt_r, mask2, hm_sums)


def kernel(hm_pred, wh_pred, reg_pred, hm_gt, reg_mask, ind, wh_gt, reg_gt):
    wh_t = wh_pred.reshape(_B * 2 * _H, _W)
    rg_t = reg_pred.reshape(_B * 2 * _H, _W)
    y = ind // _W
    x = ind - y * _W
    b_off = (jnp.arange(_B, dtype=jnp.int32) * (2 * _H))[:, None]
    rowidx = ((b_off + y)[:, :, None]
              + jnp.array([0, _H], jnp.int32)[None, None, :])
    rowidx = rowidx.reshape(_NG)
    xsel = jnp.broadcast_to(x[:, :, None], (_B, _K, 2)).reshape(_NG, 1)
    tgt_w = wh_gt.reshape(_NG, 1)
    tgt_r = reg_gt.reshape(_NG, 1)
    mask2 = jnp.broadcast_to(
        reg_mask.astype(jnp.float32)[:, :, None], (_B, _K, 2)).reshape(_NG, 1)

    gw, gr = _sc_gather2(wh_t, rg_t, rowidx)
    hm_sums = _focal_sums(hm_pred, hm_gt)
    out = _combine(gw, gr, xsel, tgt_w, tgt_r, mask2, hm_sums)
    return out[0]


# speedup vs baseline: 1.0099x; 1.0099x over previous
import functools

import jax
import jax.numpy as jnp
from jax import lax
from jax.experimental import pallas as pl
from jax.experimental.pallas import tpu as pltpu
from jax.experimental.pallas import tpu_sc as plsc

_B, _C, _H, _W, _K = 16, 80, 128, 128, 128
_HW = _H * _W
_ROWS = _B * _C * _H
_GRID = 16
_BLK = _ROWS // _GRID

_LOGIT_CLIP = 9.21024036697585

_NG = _B * _K * 2
_NW = 32
_PER_W = _NG // _NW


_CH = 8
_UNROLL = 256


def _focal_body(x_ref, g_ref, o_ref, acc_ref):
    i = pl.program_id(0)

    @pl.when(i == 0)
    def _():
        acc_ref[0] = 0.0
        acc_ref[1] = 0.0

    def step(j, carry):
        s_acc, n_acc = carry
        x = x_ref[pl.ds(j * _CH, _CH), :]
        g = g_ref[pl.ds(j * _CH, _CH), :]
        xc = jnp.minimum(jnp.maximum(x, -_LOGIT_CLIP), _LOGIT_CLIP)
        t = jnp.exp(-xc)
        u = t + 1.0
        lg = jnp.log(u)
        v = 1.0 / (u * u)
        pos = g == 1.0
        w = 1.0 - g
        w2 = w * w
        m1 = jnp.where(pos, lg * (t * t), (lg + xc) * (w2 * w2))
        s_acc = s_acc + m1 * v
        n_acc = n_acc + jnp.where(pos, 1.0, 0.0)
        return s_acc, n_acc

    z = jnp.zeros((_CH, _W), jnp.float32)
    s_acc, n_acc = lax.fori_loop(0, _BLK // _CH, step, (z, z),
                                 unroll=_UNROLL)
    acc_ref[0] += jnp.sum(s_acc)
    acc_ref[1] += jnp.sum(n_acc)

    @pl.when(i == pl.num_programs(0) - 1)
    def _():
        o_ref[0] = acc_ref[0]
        o_ref[1] = acc_ref[1]


def _focal_sums(hm_pred, hm_gt):
    x = hm_pred.reshape(_ROWS, _W)
    g = hm_gt.reshape(_ROWS, _W)
    return pl.pallas_call(
        _focal_body,
        grid=(_GRID,),
        in_specs=[
            pl.BlockSpec((_BLK, _W), lambda i: (i, 0)),
            pl.BlockSpec((_BLK, _W), lambda i: (i, 0)),
        ],
        out_specs=pl.BlockSpec(memory_space=pltpu.SMEM),
        out_shape=jax.ShapeDtypeStruct((2,), jnp.float32),
        scratch_shapes=[pltpu.SMEM((2,), jnp.float32)],
    )(x, g)


def _sc_gather2(wh_t, rg_t, idx):
    mesh = plsc.VectorSubcoreMesh(core_axis_name="c", subcore_axis_name="s")

    @functools.partial(
        pl.kernel,
        mesh=mesh,
        out_type=(jax.ShapeDtypeStruct((_NG, _W), jnp.float32),
                  jax.ShapeDtypeStruct((_NG, _W), jnp.float32)),
        scratch_types=[
            pltpu.VMEM((_PER_W,), jnp.int32),
            pltpu.VMEM((_PER_W, _W), jnp.float32),
            pltpu.VMEM((_PER_W, _W), jnp.float32),
            pltpu.SemaphoreType.DMA,
            pltpu.SemaphoreType.DMA,
        ],
    )
    def k(wh_hbm, rg_hbm, idx_hbm, ow_hbm, or_hbm,
          idx_v, rw_v, rr_v, sem1, sem2):
        wid = lax.axis_index("s") * 2 + lax.axis_index("c")
        base = wid * _PER_W
        pltpu.sync_copy(idx_hbm.at[pl.ds(base, _PER_W)], idx_v)
        c1 = pltpu.async_copy(wh_hbm.at[idx_v], rw_v, sem1)
        c2 = pltpu.async_copy(rg_hbm.at[idx_v], rr_v, sem2)
        c1.wait()
        pltpu.sync_copy(rw_v, ow_hbm.at[pl.ds(base, _PER_W)])
        c2.wait()
        pltpu.sync_copy(rr_v, or_hbm.at[pl.ds(base, _PER_W)])

    return k(wh_t, rg_t, idx)


def _combine_body(gw_ref, gr_ref, x_ref, tw_ref, tr_ref, m_ref, hm_ref, o_ref):
    onehot = (lax.broadcasted_iota(jnp.int32, (_NG, _W), 1) == x_ref[...]
              ).astype(jnp.float32)
    vw = jnp.sum(gw_ref[...] * onehot, axis=1, keepdims=True)
    vr = jnp.sum(gr_ref[...] * onehot, axis=1, keepdims=True)
    m = m_ref[...]
    contrib = (jnp.abs(vw - tw_ref[...]) * 0.1
               + jnp.abs(vr - tr_ref[...]) * 1.0) * m
    denom = jnp.sum(m) + 1e-4
    hm_loss = hm_ref[0] / jnp.maximum(hm_ref[1], 1.0)
    o_ref[0] = hm_loss + jnp.sum(contrib) / denom


def _combine(gw, gr, xsel, tgt_w, tgt_r, mask2, hm_sums):
    return pl.pallas_call(
        _combine_body,
        in_specs=[
            pl.BlockSpec(memory_space=pltpu.VMEM),
            pl.BlockSpec(memory_space=pltpu.VMEM),
            pl.BlockSpec(memory_space=pltpu.VMEM),
            pl.BlockSpec(memory_space=pltpu.VMEM),
            pl.BlockSpec(memory_space=pltpu.VMEM),
            pl.BlockSpec(memory_space=pltpu.VMEM),
            pl.BlockSpec(memory_space=pltpu.SMEM),
        ],
        out_specs=pl.BlockSpec(memory_space=pltpu.SMEM),
        out_shape=jax.ShapeDtypeStruct((1,), jnp.float32),
    )(gw, gr, xsel, tgt_w, tgt_r, mask2, hm_sums)


def kernel(hm_pred, wh_pred, reg_pred, hm_gt, reg_mask, ind, wh_gt, reg_gt):
    wh_t = wh_pred.reshape(_B * 2 * _H, _W)
    rg_t = reg_pred.reshape(_B * 2 * _H, _W)
    y = ind // _W
    x = ind - y * _W
    b_off = (jnp.arange(_B, dtype=jnp.int32) * (2 * _H))[:, None]
    rowidx = ((b_off + y)[:, :, None]
              + jnp.array([0, _H], jnp.int32)[None, None, :])
    rowidx = rowidx.reshape(_NG)
    xsel = jnp.broadcast_to(x[:, :, None], (_B, _K, 2)).reshape(_NG, 1)
    tgt_w = wh_gt.reshape(_NG, 1)
    tgt_r = reg_gt.reshape(_NG, 1)
    mask2 = jnp.broadcast_to(
        reg_mask.astype(jnp.float32)[:, :, None], (_B, _K, 2)).reshape(_NG, 1)

    gw, gr = _sc_gather2(wh_t, rg_t, rowidx)
    hm_sums = _focal_sums(hm_pred, hm_gt)
    out = _combine(gw, gr, xsel, tgt_w, tgt_r, mask2, hm_sums)
    return out[0]


# speedup vs baseline: 1.0109x; 1.0010x over previous
import functools

import jax
import jax.numpy as jnp
from jax import lax
from jax.experimental import pallas as pl
from jax.experimental.pallas import tpu as pltpu
from jax.experimental.pallas import tpu_sc as plsc

_B, _C, _H, _W, _K = 16, 80, 128, 128, 128
_HW = _H * _W
_ROWS = _B * _C * _H
_GRID = 16
_BLK = _ROWS // _GRID

_LOGIT_CLIP = 9.21024036697585

_NG = _B * _K * 2
_NW = 32
_PER_W = _NG // _NW


_CH = 16
_UNROLL = 128


def _focal_body(x_ref, g_ref, o_ref, acc_ref):
    i = pl.program_id(0)

    @pl.when(i == 0)
    def _():
        acc_ref[0] = 0.0
        acc_ref[1] = 0.0

    def step(j, carry):
        s_acc, n_acc = carry
        x = x_ref[pl.ds(j * _CH, _CH), :]
        g = g_ref[pl.ds(j * _CH, _CH), :]
        xc = jnp.minimum(jnp.maximum(x, -_LOGIT_CLIP), _LOGIT_CLIP)
        t = jnp.exp(-xc)
        u = t + 1.0
        lg = jnp.log(u)
        v = 1.0 / (u * u)
        pos = g == 1.0
        w = 1.0 - g
        w2 = w * w
        m1 = jnp.where(pos, lg * (t * t), (lg + xc) * (w2 * w2))
        s_acc = s_acc + m1 * v
        n_acc = n_acc + jnp.where(pos, 1.0, 0.0)
        return s_acc, n_acc

    z = jnp.zeros((_CH, _W), jnp.float32)
    s_acc, n_acc = lax.fori_loop(0, _BLK // _CH, step, (z, z),
                                 unroll=_UNROLL)
    acc_ref[0] += jnp.sum(s_acc)
    acc_ref[1] += jnp.sum(n_acc)

    @pl.when(i == pl.num_programs(0) - 1)
    def _():
        o_ref[0] = acc_ref[0]
        o_ref[1] = acc_ref[1]


def _focal_sums(hm_pred, hm_gt):
    x = hm_pred.reshape(_ROWS, _W)
    g = hm_gt.reshape(_ROWS, _W)
    return pl.pallas_call(
        _focal_body,
        grid=(_GRID,),
        in_specs=[
            pl.BlockSpec((_BLK, _W), lambda i: (i, 0)),
            pl.BlockSpec((_BLK, _W), lambda i: (i, 0)),
        ],
        out_specs=pl.BlockSpec(memory_space=pltpu.SMEM),
        out_shape=jax.ShapeDtypeStruct((2,), jnp.float32),
        scratch_shapes=[pltpu.SMEM((2,), jnp.float32)],
    )(x, g)


def _sc_gather2(wh_t, rg_t, idx):
    mesh = plsc.VectorSubcoreMesh(core_axis_name="c", subcore_axis_name="s")

    @functools.partial(
        pl.kernel,
        mesh=mesh,
        out_type=(jax.ShapeDtypeStruct((_NG, _W), jnp.float32),
                  jax.ShapeDtypeStruct((_NG, _W), jnp.float32)),
        scratch_types=[
            pltpu.VMEM((_PER_W,), jnp.int32),
            pltpu.VMEM((_PER_W, _W), jnp.float32),
            pltpu.VMEM((_PER_W, _W), jnp.float32),
            pltpu.SemaphoreType.DMA,
            pltpu.SemaphoreType.DMA,
        ],
    )
    def k(wh_hbm, rg_hbm, idx_hbm, ow_hbm, or_hbm,
          idx_v, rw_v, rr_v, sem1, sem2):
        wid = lax.axis_index("s") * 2 + lax.axis_index("c")
        base = wid * _PER_W
        pltpu.sync_copy(idx_hbm.at[pl.ds(base, _PER_W)], idx_v)
        c1 = pltpu.async_copy(wh_hbm.at[idx_v], rw_v, sem1)
        c2 = pltpu.async_copy(rg_hbm.at[idx_v], rr_v, sem2)
        c1.wait()
        pltpu.sync_copy(rw_v, ow_hbm.at[pl.ds(base, _PER_W)])
        c2.wait()
        pltpu.sync_copy(rr_v, or_hbm.at[pl.ds(base, _PER_W)])

    return k(wh_t, rg_t, idx)


def _combine_body(gw_ref, gr_ref, x_ref, tw_ref, tr_ref, m_ref, hm_ref, o_ref):
    onehot = (lax.broadcasted_iota(jnp.int32, (_NG, _W), 1) == x_ref[...]
              ).astype(jnp.float32)
    vw = jnp.sum(gw_ref[...] * onehot, axis=1, keepdims=True)
    vr = jnp.sum(gr_ref[...] * onehot, axis=1, keepdims=True)
    m = m_ref[...]
    contrib = (jnp.abs(vw - tw_ref[...]) * 0.1
               + jnp.abs(vr - tr_ref[...]) * 1.0) * m
    denom = jnp.sum(m) + 1e-4
    hm_loss = hm_ref[0] / jnp.maximum(hm_ref[1], 1.0)
    o_ref[0] = hm_loss + jnp.sum(contrib) / denom


def _combine(gw, gr, xsel, tgt_w, tgt_r, mask2, hm_sums):
    return pl.pallas_call(
        _combine_body,
        in_specs=[
            pl.BlockSpec(memory_space=pltpu.VMEM),
            pl.BlockSpec(memory_space=pltpu.VMEM),
            pl.BlockSpec(memory_space=pltpu.VMEM),
            pl.BlockSpec(memory_space=pltpu.VMEM),
            pl.BlockSpec(memory_space=pltpu.VMEM),
            pl.BlockSpec(memory_space=pltpu.VMEM),
            pl.BlockSpec(memory_space=pltpu.SMEM),
        ],
        out_specs=pl.BlockSpec(memory_space=pltpu.SMEM),
        out_shape=jax.ShapeDtypeStruct((1,), jnp.float32),
    )(gw, gr, xsel, tgt_w, tgt_r, mask2, hm_sums)


def kernel(hm_pred, wh_pred, reg_pred, hm_gt, reg_mask, ind, wh_gt, reg_gt):
    wh_t = wh_pred.reshape(_B * 2 * _H, _W)
    rg_t = reg_pred.reshape(_B * 2 * _H, _W)
    y = ind // _W
    x = ind - y * _W
    b_off = (jnp.arange(_B, dtype=jnp.int32) * (2 * _H))[:, None]
    rowidx = ((b_off + y)[:, :, None]
              + jnp.array([0, _H], jnp.int32)[None, None, :])
    rowidx = rowidx.reshape(_NG)
    xsel = jnp.broadcast_to(x[:, :, None], (_B, _K, 2)).reshape(_NG, 1)
    tgt_w = wh_gt.reshape(_NG, 1)
    tgt_r = reg_gt.reshape(_NG, 1)
    mask2 = jnp.broadcast_to(
        reg_mask.astype(jnp.float32)[:, :, None], (_B, _K, 2)).reshape(_NG, 1)

    gw, gr = _sc_gather2(wh_t, rg_t, rowidx)
    hm_sums = _focal_sums(hm_pred, hm_gt)
    out = _combine(gw, gr, xsel, tgt_w, tgt_r, mask2, hm_sums)
    return out[0]
